# baseline (speedup 1.0000x reference)
import functools

import jax
import jax.numpy as jnp
from jax import lax
from jax.experimental import pallas as pl
from jax.experimental.pallas import tpu as pltpu
from jax.experimental.pallas import tpu_sc as plsc

NC = 2
NS = 16
NW = NC * NS
L = 16

B = 16384
D = 64
BPW = B // NW
CH = 128
NCH = BPW // CH
GRP = BPW // L


def _sc_body(hidx, ridx, tidx, rel_hbm, ent_hbm, out_hbm,
             idx_h, idx_r, idx_t, rows_h, rows_r, rows_t, tpose, out_v, sem):
    c = lax.axis_index("c")
    s = lax.axis_index("s")
    w = s * NC + c

    pltpu.sync_copy(hidx.at[w], idx_h)
    pltpu.sync_copy(ridx.at[w], idx_r)
    pltpu.sync_copy(tidx.at[w], idx_t)

    copies = []
    for j in range(NCH):
        copies.append(pltpu.async_copy(
            ent_hbm.at[idx_h.at[j]], rows_h.at[pl.ds(j * CH, CH)], sem))
        copies.append(pltpu.async_copy(
            rel_hbm.at[idx_r.at[j]], rows_r.at[pl.ds(j * CH, CH)], sem))
        copies.append(pltpu.async_copy(
            ent_hbm.at[idx_t.at[j]], rows_t.at[pl.ds(j * CH, CH)], sem))
    for cp in copies:
        cp.wait()

    lanes = lax.iota(jnp.int32, L)

    def group(g, carry):
        score = jnp.zeros((L,), jnp.float32)
        for s16 in range(L):
            row = g * L + s16
            acc = None
            for k in range(D // L):
                h = rows_h[row, pl.ds(k * L, L)]
                r = rows_r[row, pl.ds(k * L, L)]
                t = rows_t[row, pl.ds(k * L, L)]
                p = h * r * t
                acc = p if acc is None else acc + p
            score = jnp.where(lanes == s16, jnp.sum(acc), score)
        out_v[pl.ds(g * L, L)] = score
        return carry

    lax.fori_loop(0, GRP, group, 0)

    pltpu.sync_copy(out_v, out_hbm.at[pl.ds(w * BPW, BPW)])


@functools.partial(jax.jit, static_argnames=())
def _sc_call(hidx, ridx, tidx, rel, ent):
    mesh = plsc.VectorSubcoreMesh(core_axis_name="c", subcore_axis_name="s")
    f = functools.partial(
        pl.kernel,
        out_type=jax.ShapeDtypeStruct((B,), jnp.float32),
        mesh=mesh,
        compiler_params=pltpu.CompilerParams(
            needs_layout_passes=False, use_tc_tiling_on_sc=False),
        scratch_types=[
            pltpu.VMEM((NCH, CH), jnp.int32),
            pltpu.VMEM((NCH, CH), jnp.int32),
            pltpu.VMEM((NCH, CH), jnp.int32),
            pltpu.VMEM((BPW, D), jnp.float32),
            pltpu.VMEM((BPW, D), jnp.float32),
            pltpu.VMEM((BPW, D), jnp.float32),
            pltpu.VMEM((L, L), jnp.float32),
            pltpu.VMEM((BPW,), jnp.float32),
            pltpu.SemaphoreType.DMA,
        ],
    )(_sc_body)
    return f(hidx, ridx, tidx, rel, ent)


def kernel(sample, relation_embedding, entity_embedding, neg):
    sample = sample.astype(jnp.int32)
    hidx = sample[:, 0].reshape(NW, NCH, CH)
    ridx = sample[:, 1].reshape(NW, NCH, CH)
    tidx = sample[:, 2].reshape(NW, NCH, CH)
    out = _sc_call(hidx, ridx, tidx, relation_embedding, entity_embedding)
    return out.reshape(B, 1)
